# baseline (speedup 1.0000x reference)
import jax
import jax.numpy as jnp
from jax import lax
from jax.experimental import pallas as pl
from jax.experimental.pallas import tpu as pltpu
from jax.experimental.pallas import tpu_sc as plsc

M = 16
MAX_NORM = 1.0 - 1e-05
MAX_NORM_SQ = MAX_NORM * MAX_NORM

NC = 2
NS = 16
NW = NC * NS
L = 16

B = 16384
F = 26
B_TOTAL = B * F
B_PER_W = B_TOTAL // NW
BLK = 128
NBLK_W = B_PER_W // BLK
NBLK_B = B // BLK
NSC = 8
BLK_SC = NBLK_W // NSC
C = BLK_SC * BLK
GROUPS = BLK // L
OUT_LEN = F * 2 * NBLK_B * 1024

N_ROWS = 1000000
TW = 12800
TSTEPS = -(-N_ROWS // TW)


def _transpose_body(i_ref, o_ref):
    x = i_ref[...]
    t = jnp.transpose(x.reshape(M, TW // 8, 8), (1, 2, 0))
    o_ref[...] = t.reshape(TW // 8, 128)


@jax.jit
def _to_row_major(w_t):
    return pl.pallas_call(
        _transpose_body,
        grid=(TSTEPS,),
        in_specs=[pl.BlockSpec((M, TW), lambda i: (0, i))],
        out_specs=pl.BlockSpec((TW // 8, 128), lambda i: (i, 0)),
        out_shape=jax.ShapeDtypeStruct((N_ROWS // 8, 128), jnp.float32),
    )(w_t)


def _rsqrt16(ss):
    bits = lax.bitcast_convert_type(ss, jnp.int32)
    y = lax.bitcast_convert_type(
        jnp.int32(0x5F3759DF) - lax.shift_right_arithmetic(bits, 1),
        jnp.float32)
    for _ in range(3):
        y = y * (1.5 - 0.5 * ss * y * y)
    return y


def _sc_kernel(w_hbm, idx_hbm, out_hbm, idx_v, rows_a, rows_b, outt,
               sem_a, sem_b, sem_oa, sem_ob):
    wid = lax.axis_index("s") * NC + lax.axis_index("c")
    base = wid * B_PER_W
    iota = lax.iota(jnp.int32, L)

    pltpu.sync_copy(idx_hbm.at[pl.ds(base, B_PER_W)], idx_v)

    rows = (rows_a, rows_b)
    gsems = (sem_a, sem_b)
    osems = (sem_oa, sem_ob)

    def gather(sc):
        b = sc % 2
        return pltpu.async_copy(
            w_hbm.at[idx_v.at[pl.ds(sc * C, C)]], rows[b], gsems[b])

    def compute_sc(sc):
        cur = sc % 2
        buf = rows[cur]

        def body(blk, carry):
            for g in range(GROUPS):
                row_ids = blk * BLK + g * L + iota
                cols = [
                    plsc.load_gather(
                        buf, [row_ids, jnp.full((L,), j, jnp.int32)])
                    for j in range(M)
                ]
                ss = cols[0] * cols[0]
                for j in range(1, M):
                    ss = ss + cols[j] * cols[j]
                scale = jnp.where(ss > MAX_NORM_SQ, MAX_NORM * _rsqrt16(ss),
                                  jnp.float32(1.0))
                for j in range(M):
                    outt[cur, j // 8, blk,
                         pl.ds((j % 8) * BLK + g * L, L)] = cols[j] * scale
            return carry

        lax.fori_loop(0, BLK_SC, body, 0, unroll=False)

    def emit_out(sc):
        cur = sc % 2
        handles = []
        for blk in range(BLK_SC):
            gblk = wid * NBLK_W + sc * BLK_SC + blk
            f26 = gblk // NBLK_B
            cpos = gblk % NBLK_B
            for band in range(2):
                off = (f26 * 256 + band * NBLK_B + cpos) * 1024
                off = pl.multiple_of(off, 1024)
                handles.append(pltpu.async_copy(
                    outt.at[cur, band, blk],
                    out_hbm.at[pl.ds(off, 1024)], osems[cur]))
        return handles

    pending_out = [None, None]
    ghandles = {0: gather(0)}
    for sc in range(NSC):
        b = sc % 2
        if sc + 1 < NSC:
            ghandles[sc + 1] = gather(sc + 1)
        ghandles.pop(sc).wait()
        if pending_out[b] is not None:
            for h in pending_out[b]:
                h.wait()
        compute_sc(sc)
        pending_out[b] = emit_out(sc)
    for par in pending_out:
        if par is not None:
            for h in par:
                h.wait()


@jax.jit
def _run(idx_flat, weight_rm):
    mesh = plsc.VectorSubcoreMesh(core_axis_name="c", subcore_axis_name="s")
    f = pl.kernel(
        _sc_kernel,
        out_type=jax.ShapeDtypeStruct((OUT_LEN,), jnp.float32),
        mesh=mesh,
        compiler_params=pltpu.CompilerParams(
            needs_layout_passes=False, use_tc_tiling_on_sc=False),
        scratch_types=[
            pltpu.VMEM((B_PER_W,), jnp.int32),
            pltpu.VMEM((C, M), jnp.float32),
            pltpu.VMEM((C, M), jnp.float32),
            pltpu.VMEM((2, 2, BLK_SC, 1024), jnp.float32),
            pltpu.SemaphoreType.DMA,
            pltpu.SemaphoreType.DMA,
            pltpu.SemaphoreType.DMA,
            pltpu.SemaphoreType.DMA,
        ],
    )
    return f(weight_rm, idx_flat)


def kernel(x, weight):
    idx_flat = x.T.reshape(-1).astype(jnp.int32)
    w128 = _to_row_major(weight.T)
    w_rm = w128.reshape(N_ROWS * M).reshape(N_ROWS, M)
    out_flat = _run(idx_flat, w_rm)
    out5 = out_flat.reshape(F, 2, NBLK_B, 8, BLK)
    return out5.transpose(2, 4, 0, 1, 3).reshape(B, F, M)


# speedup vs baseline: 3.9846x; 3.9846x over previous
import jax
import jax.numpy as jnp
from jax import lax
from jax.experimental import pallas as pl
from jax.experimental.pallas import tpu as pltpu
from jax.experimental.pallas import tpu_sc as plsc

M = 16
MAX_NORM = 1.0 - 1e-05
MAX_NORM_SQ = MAX_NORM * MAX_NORM

NC = 2
NS = 16
NW = NC * NS
L = 16

B = 16384
F = 26
B_TOTAL = B * F
B_PER_W = B_TOTAL // NW
BLK = 128
NBLK_W = B_PER_W // BLK
NBLK_B = B // BLK
GROUPS = BLK // L
WROW = 128
OUT_LEN = F * 2 * NBLK_B * 1024

N_ROWS = 1000000
TCHUNK = 1024
TQ = 8
TW = TCHUNK * TQ
TSTEPS = -(-N_ROWS // TW)
W_BLK_ROWS = TSTEPS * TQ * 128


def _repack_body(i_ref, o_ref):
    xb = i_ref[...]
    tiles = []
    for q in range(TQ):
        stacked = jnp.concatenate(
            [xb[:, q * TCHUNK + g * 128:q * TCHUNK + (g + 1) * 128]
             for g in range(8)], axis=0)
        tiles.append(stacked.T)
    o_ref[...] = jnp.concatenate(tiles, axis=0)


@jax.jit
def _to_blocked(w_t):
    return pl.pallas_call(
        _repack_body,
        grid=(TSTEPS,),
        in_specs=[pl.BlockSpec((M, TW), lambda i: (0, i))],
        out_specs=pl.BlockSpec((TQ * 128, 128), lambda i: (i, 0)),
        out_shape=jax.ShapeDtypeStruct((W_BLK_ROWS, 128), jnp.float32),
    )(w_t)


def _rsqrt16(ss):
    bits = lax.bitcast_convert_type(ss, jnp.int32)
    y = lax.bitcast_convert_type(
        jnp.int32(0x5F3759DF) - lax.shift_right_arithmetic(bits, 1),
        jnp.float32)
    for _ in range(3):
        y = y * (1.5 - 0.5 * ss * y * y)
    return y


def _sc_kernel(w_hbm, idx_hbm, out_hbm, idx_v, blk_v, coff_v, rows2, outt,
               sem_g, sem_o):
    wid = lax.axis_index("s") * NC + lax.axis_index("c")
    base = wid * B_PER_W
    iota = lax.iota(jnp.int32, L)

    pltpu.sync_copy(idx_hbm.at[pl.ds(base, B_PER_W)], idx_v)

    def split_idx(i, carry):
        v = idx_v[pl.ds(i * L, L)]
        blk_v[pl.ds(i * L, L)] = lax.shift_left(
            lax.shift_right_logical(v, 10), 7) | (v & 127)
        coff_v[pl.ds(i * L, L)] = lax.shift_left(
            lax.shift_right_logical(v, 7) & 7, 4)
        return carry

    lax.fori_loop(0, B_PER_W // L, split_idx, 0, unroll=False)

    def gather(b):
        par = b & 1
        pltpu.async_copy(
            w_hbm.at[blk_v.at[pl.ds(b * BLK, BLK)]],
            rows2.at[pl.ds(par * BLK, BLK)], sem_g)

    def wait_gather():
        pltpu.make_async_copy(
            w_hbm.at[pl.ds(0, BLK)], rows2.at[pl.ds(0, BLK)], sem_g).wait()

    def drain_out():
        pltpu.make_async_copy(
            out_hbm.at[pl.ds(0, 1024)], outt.at[0, 0], sem_o).wait()

    gather(0)

    def body(b, carry):
        par = b & 1
        pl.when(b + 1 < NBLK_W)(lambda: gather(b + 1))
        wait_gather()

        @pl.when(b >= 2)
        def _():
            drain_out()
            drain_out()

        for g in range(GROUPS):
            rid = par * BLK + g * L + iota
            coff = coff_v[pl.ds(b * BLK + g * L, L)]
            cols = [plsc.load_gather(rows2, [rid, coff + j]) for j in range(M)]
            ss = cols[0] * cols[0]
            for j in range(1, M):
                ss = ss + cols[j] * cols[j]
            scale = jnp.where(ss > MAX_NORM_SQ, MAX_NORM * _rsqrt16(ss),
                              jnp.float32(1.0))
            for j in range(M):
                outt[par, j // 8, pl.ds((j % 8) * BLK + g * L, L)] = (
                    cols[j] * scale)
        gblk = wid * NBLK_W + b
        f26 = gblk // NBLK_B
        cpos = gblk % NBLK_B
        for band in range(2):
            off = (f26 * 256 + band * NBLK_B + cpos) * 1024
            off = pl.multiple_of(off, 1024)
            pltpu.async_copy(outt.at[par, band],
                             out_hbm.at[pl.ds(off, 1024)], sem_o)
        return carry

    lax.fori_loop(0, NBLK_W, body, 0, unroll=False)
    for _ in range(4):
        drain_out()


@jax.jit
def _run(idx_flat, w_blk):
    mesh = plsc.VectorSubcoreMesh(core_axis_name="c", subcore_axis_name="s")
    f = pl.kernel(
        _sc_kernel,
        out_type=jax.ShapeDtypeStruct((OUT_LEN,), jnp.float32),
        mesh=mesh,
        compiler_params=pltpu.CompilerParams(
            needs_layout_passes=False, use_tc_tiling_on_sc=False),
        scratch_types=[
            pltpu.VMEM((B_PER_W,), jnp.int32),
            pltpu.VMEM((B_PER_W,), jnp.int32),
            pltpu.VMEM((B_PER_W,), jnp.int32),
            pltpu.VMEM((2 * BLK, WROW), jnp.float32),
            pltpu.VMEM((2, 2, 1024), jnp.float32),
            pltpu.SemaphoreType.DMA,
            pltpu.SemaphoreType.DMA,
        ],
    )
    return f(w_blk, idx_flat)


def kernel(x, weight):
    idx_flat = x.T.reshape(-1).astype(jnp.int32)
    w_blk = _to_blocked(weight.T)
    out_flat = _run(idx_flat, w_blk)
    out5 = out_flat.reshape(F, 2, NBLK_B, 8, BLK)
    return out5.transpose(2, 4, 0, 1, 3).reshape(B, F, M)


# speedup vs baseline: 4.8236x; 1.2106x over previous
import jax
import jax.numpy as jnp
from jax import lax
from jax.experimental import pallas as pl
from jax.experimental.pallas import tpu as pltpu
from jax.experimental.pallas import tpu_sc as plsc

M = 16
MAX_NORM = 1.0 - 1e-05
MAX_NORM_SQ = MAX_NORM * MAX_NORM

NC = 2
NS = 16
NW = NC * NS
L = 16

B = 16384
F = 26
B_TOTAL = B * F
B_PER_W = B_TOTAL // NW
BLK = 128
NBLK_W = B_PER_W // BLK
NBLK_B = B // BLK
NSC = 8
BLK_SC = NBLK_W // NSC
C = BLK_SC * BLK
GROUPS = BLK // L
OUT_LEN = F * 2 * NBLK_B * 1024

N_ROWS = 1000000
TCHUNK = 1024
TQ = 8
TW = TCHUNK * TQ
TSTEPS = -(-N_ROWS // TW)
W_BLK_ROWS = TSTEPS * TQ * 128
W16_ROWS = W_BLK_ROWS * 8


def _repack_body(i_ref, o_ref):
    xb = i_ref[...]
    tiles = []
    for q in range(TQ):
        stacked = jnp.concatenate(
            [xb[:, q * TCHUNK + g * 128:q * TCHUNK + (g + 1) * 128]
             for g in range(8)], axis=0)
        tiles.append(stacked.T)
    o_ref[...] = jnp.concatenate(tiles, axis=0)


@jax.jit
def _to_blocked(w_t):
    return pl.pallas_call(
        _repack_body,
        grid=(TSTEPS,),
        in_specs=[pl.BlockSpec((M, TW), lambda i: (0, i))],
        out_specs=pl.BlockSpec((TQ * 128, 128), lambda i: (i, 0)),
        out_shape=jax.ShapeDtypeStruct((W_BLK_ROWS, 128), jnp.float32),
    )(w_t)


def _rsqrt16(ss):
    bits = lax.bitcast_convert_type(ss, jnp.int32)
    y = lax.bitcast_convert_type(
        jnp.int32(0x5F3759DF) - lax.shift_right_arithmetic(bits, 1),
        jnp.float32)
    for _ in range(3):
        y = y * (1.5 - 0.5 * ss * y * y)
    return y


def _sc_kernel(w_hbm, idx_hbm, out_hbm, idx_v, rows_a, rows_b, outt,
               sem_a, sem_b, sem_oa, sem_ob):
    wid = lax.axis_index("s") * NC + lax.axis_index("c")
    base = wid * B_PER_W
    iota = lax.iota(jnp.int32, L)

    pltpu.sync_copy(idx_hbm.at[pl.ds(base, B_PER_W)], idx_v)

    def remap_idx(i, carry):
        v = idx_v[pl.ds(i * L, L)]
        idx_v[pl.ds(i * L, L)] = (
            lax.shift_left(lax.shift_right_logical(v, 10), 10)
            | lax.shift_left(v & 127, 3)
            | (lax.shift_right_logical(v, 7) & 7))
        return carry

    lax.fori_loop(0, B_PER_W // L, remap_idx, 0, unroll=False)

    rows = (rows_a, rows_b)
    gsems = (sem_a, sem_b)
    osems = (sem_oa, sem_ob)

    def gather(sc):
        b = sc % 2
        return pltpu.async_copy(
            w_hbm.at[idx_v.at[pl.ds(sc * C, C)]], rows[b], gsems[b])

    def compute_sc(sc):
        cur = sc % 2
        buf = rows[cur]

        def body(blk, carry):
            for g in range(GROUPS):
                row_ids = blk * BLK + g * L + iota
                cols = [
                    plsc.load_gather(
                        buf, [row_ids, jnp.full((L,), j, jnp.int32)])
                    for j in range(M)
                ]
                ss = cols[0] * cols[0]
                for j in range(1, M):
                    ss = ss + cols[j] * cols[j]
                scale = jnp.where(ss > MAX_NORM_SQ, MAX_NORM * _rsqrt16(ss),
                                  jnp.float32(1.0))
                for j in range(M):
                    outt[cur, j // 8, blk,
                         pl.ds((j % 8) * BLK + g * L, L)] = cols[j] * scale
            return carry

        lax.fori_loop(0, BLK_SC, body, 0, unroll=False)

    def emit_out(sc):
        cur = sc % 2
        handles = []
        for blk in range(BLK_SC):
            gblk = wid * NBLK_W + sc * BLK_SC + blk
            f26 = gblk // NBLK_B
            cpos = gblk % NBLK_B
            for band in range(2):
                off = (f26 * 256 + band * NBLK_B + cpos) * 1024
                off = pl.multiple_of(off, 1024)
                handles.append(pltpu.async_copy(
                    outt.at[cur, band, blk],
                    out_hbm.at[pl.ds(off, 1024)], osems[cur]))
        return handles

    pending_out = [None, None]
    ghandles = {0: gather(0)}
    for sc in range(NSC):
        b = sc % 2
        if sc + 1 < NSC:
            ghandles[sc + 1] = gather(sc + 1)
        ghandles.pop(sc).wait()
        if pending_out[b] is not None:
            for h in pending_out[b]:
                h.wait()
        compute_sc(sc)
        pending_out[b] = emit_out(sc)
    for par in pending_out:
        if par is not None:
            for h in par:
                h.wait()


@jax.jit
def _run(idx_flat, w16):
    mesh = plsc.VectorSubcoreMesh(core_axis_name="c", subcore_axis_name="s")
    f = pl.kernel(
        _sc_kernel,
        out_type=jax.ShapeDtypeStruct((OUT_LEN,), jnp.float32),
        mesh=mesh,
        compiler_params=pltpu.CompilerParams(
            needs_layout_passes=False, use_tc_tiling_on_sc=False),
        scratch_types=[
            pltpu.VMEM((B_PER_W,), jnp.int32),
            pltpu.VMEM((C, M), jnp.float32),
            pltpu.VMEM((C, M), jnp.float32),
            pltpu.VMEM((2, 2, BLK_SC, 1024), jnp.float32),
            pltpu.SemaphoreType.DMA,
            pltpu.SemaphoreType.DMA,
            pltpu.SemaphoreType.DMA,
            pltpu.SemaphoreType.DMA,
        ],
    )
    return f(w16, idx_flat)


def kernel(x, weight):
    idx_flat = x.T.reshape(-1).astype(jnp.int32)
    w_blk = _to_blocked(weight.T)
    w16 = w_blk.reshape(W16_ROWS * M).reshape(W16_ROWS, M)
    out_flat = _run(idx_flat, w16)
    out5 = out_flat.reshape(F, 2, NBLK_B, 8, BLK)
    return out5.transpose(2, 4, 0, 1, 3).reshape(B, F, M)


# speedup vs baseline: 5.6093x; 1.1629x over previous
import jax
import jax.numpy as jnp
from jax import lax
from jax.experimental import pallas as pl
from jax.experimental.pallas import tpu as pltpu
from jax.experimental.pallas import tpu_sc as plsc

M = 16
MAX_NORM = 1.0 - 1e-05
MAX_NORM_SQ = MAX_NORM * MAX_NORM

NC = 2
NS = 16
NW = NC * NS
L = 16

B = 16384
F = 26
B_TOTAL = B * F
B_PER_W = B_TOTAL // NW
BLK = 128
NBLK_W = B_PER_W // BLK
NBLK_B = B // BLK
NSC = 8
BLK_SC = NBLK_W // NSC
C = BLK_SC * BLK
GROUPS = BLK // L
OUT_LEN = F * 2 * NBLK_B * 1024

N_ROWS = 1000000
TCHUNK = 1024
TQ = 16
TW = TCHUNK * TQ
TSTEPS = -(-N_ROWS // TW)
W_BLK_ROWS = TSTEPS * TQ * 128
W16_ROWS = W_BLK_ROWS * 8


def _repack_body(i_ref, o_ref):
    xb = i_ref[...]
    tiles = []
    for q in range(TQ):
        stacked = jnp.concatenate(
            [xb[:, q * TCHUNK + g * 128:q * TCHUNK + (g + 1) * 128]
             for g in range(8)], axis=0)
        tiles.append(stacked.T)
    o_ref[...] = jnp.concatenate(tiles, axis=0)


@jax.jit
def _to_blocked(w_t):
    return pl.pallas_call(
        _repack_body,
        grid=(TSTEPS,),
        in_specs=[pl.BlockSpec((M, TW), lambda i: (0, i))],
        out_specs=pl.BlockSpec((TQ * 128, 128), lambda i: (i, 0)),
        out_shape=jax.ShapeDtypeStruct((W_BLK_ROWS, 128), jnp.float32),
    )(w_t)


def _rsqrt16(ss):
    bits = lax.bitcast_convert_type(ss, jnp.int32)
    y = lax.bitcast_convert_type(
        jnp.int32(0x5F3759DF) - lax.shift_right_arithmetic(bits, 1),
        jnp.float32)
    for _ in range(3):
        y = y * (1.5 - 0.5 * ss * y * y)
    return y


def _sc_kernel(w_hbm, idx_hbm, out_hbm, idx_v, rows_a, rows_b, outt,
               sem_a, sem_b, sem_oa, sem_ob):
    wid = lax.axis_index("s") * NC + lax.axis_index("c")
    base = wid * B_PER_W
    iota = lax.iota(jnp.int32, L)

    pltpu.sync_copy(idx_hbm.at[pl.ds(base, B_PER_W)], idx_v)

    def remap_idx(i, carry):
        v = idx_v[pl.ds(i * L, L)]
        idx_v[pl.ds(i * L, L)] = (
            lax.shift_left(lax.shift_right_logical(v, 10), 10)
            | lax.shift_left(v & 127, 3)
            | (lax.shift_right_logical(v, 7) & 7))
        return carry

    lax.fori_loop(0, B_PER_W // L, remap_idx, 0, unroll=False)

    rows = (rows_a, rows_b)
    gsems = (sem_a, sem_b)
    osems = (sem_oa, sem_ob)

    def gather(sc):
        b = sc % 2
        return pltpu.async_copy(
            w_hbm.at[idx_v.at[pl.ds(sc * C, C)]], rows[b], gsems[b])

    def compute_sc(sc):
        cur = sc % 2
        buf = rows[cur]

        def body(blk, carry):
            for g in range(GROUPS):
                row_ids = blk * BLK + g * L + iota
                cols = [
                    plsc.load_gather(
                        buf, [row_ids, jnp.full((L,), j, jnp.int32)])
                    for j in range(M)
                ]
                ss = cols[0] * cols[0]
                for j in range(1, M):
                    ss = ss + cols[j] * cols[j]
                scale = jnp.where(ss > MAX_NORM_SQ, MAX_NORM * _rsqrt16(ss),
                                  jnp.float32(1.0))
                for j in range(M):
                    outt[cur, j // 8, blk,
                         pl.ds((j % 8) * BLK + g * L, L)] = cols[j] * scale
            return carry

        lax.fori_loop(0, BLK_SC, body, 0, unroll=False)

    def emit_out(sc):
        cur = sc % 2
        handles = []
        for blk in range(BLK_SC):
            gblk = wid * NBLK_W + sc * BLK_SC + blk
            f26 = gblk // NBLK_B
            cpos = gblk % NBLK_B
            for band in range(2):
                off = (f26 * 256 + band * NBLK_B + cpos) * 1024
                off = pl.multiple_of(off, 1024)
                handles.append(pltpu.async_copy(
                    outt.at[cur, band, blk],
                    out_hbm.at[pl.ds(off, 1024)], osems[cur]))
        return handles

    pending_out = [None, None]
    ghandles = {0: gather(0)}
    for sc in range(NSC):
        b = sc % 2
        if sc + 1 < NSC:
            ghandles[sc + 1] = gather(sc + 1)
        ghandles.pop(sc).wait()
        if pending_out[b] is not None:
            for h in pending_out[b]:
                h.wait()
        compute_sc(sc)
        pending_out[b] = emit_out(sc)
    for par in pending_out:
        if par is not None:
            for h in par:
                h.wait()


@jax.jit
def _run(idx_flat, w16):
    mesh = plsc.VectorSubcoreMesh(core_axis_name="c", subcore_axis_name="s")
    f = pl.kernel(
        _sc_kernel,
        out_type=jax.ShapeDtypeStruct((OUT_LEN,), jnp.float32),
        mesh=mesh,
        compiler_params=pltpu.CompilerParams(
            needs_layout_passes=False, use_tc_tiling_on_sc=False),
        scratch_types=[
            pltpu.VMEM((B_PER_W,), jnp.int32),
            pltpu.VMEM((C, M), jnp.float32),
            pltpu.VMEM((C, M), jnp.float32),
            pltpu.VMEM((2, 2, BLK_SC, 1024), jnp.float32),
            pltpu.SemaphoreType.DMA,
            pltpu.SemaphoreType.DMA,
            pltpu.SemaphoreType.DMA,
            pltpu.SemaphoreType.DMA,
        ],
    )
    return f(w16, idx_flat)


def kernel(x, weight):
    idx_flat = x.T.reshape(-1).astype(jnp.int32)
    w_blk = _to_blocked(weight.T)
    w16 = w_blk.reshape(W16_ROWS * M).reshape(W16_ROWS, M)
    out_flat = _run(idx_flat, w16)
    out5 = out_flat.reshape(F, 2, NBLK_B, 8, BLK)
    return out5.transpose(2, 4, 0, 1, 3).reshape(B, F, M)


# speedup vs baseline: 6.2848x; 1.1204x over previous
import jax
import jax.numpy as jnp
from jax import lax
from jax.experimental import pallas as pl
from jax.experimental.pallas import tpu as pltpu
from jax.experimental.pallas import tpu_sc as plsc

M = 16
MAX_NORM = 1.0 - 1e-05
MAX_NORM_SQ = MAX_NORM * MAX_NORM

NC = 2
NS = 16
NW = NC * NS
L = 16

B = 16384
F = 26
B_TOTAL = B * F
B_PER_W = B_TOTAL // NW
BLK = 128
NBLK_W = B_PER_W // BLK
NBLK_B = B // BLK
NSC = 8
BLK_SC = NBLK_W // NSC
C = BLK_SC * BLK
GROUPS = BLK // L
OUT_LEN = F * 2 * NBLK_B * 1024

N_ROWS = 1000000
TCHUNK = 1024
TQ = 32
TW = TCHUNK * TQ
TSTEPS = -(-N_ROWS // TW)
W_BLK_ROWS = TSTEPS * TQ * 128
W16_ROWS = W_BLK_ROWS * 8


def _repack_body(i_ref, o_ref):
    xb = i_ref[...]
    tiles = []
    for q in range(TQ):
        stacked = jnp.concatenate(
            [xb[:, q * TCHUNK + g * 128:q * TCHUNK + (g + 1) * 128]
             for g in range(8)], axis=0)
        tiles.append(stacked.T)
    o_ref[...] = jnp.concatenate(tiles, axis=0)


@jax.jit
def _to_blocked(w_t):
    return pl.pallas_call(
        _repack_body,
        grid=(TSTEPS,),
        in_specs=[pl.BlockSpec((M, TW), lambda i: (0, i))],
        out_specs=pl.BlockSpec((TQ * 128, 128), lambda i: (i, 0)),
        out_shape=jax.ShapeDtypeStruct((W_BLK_ROWS, 128), jnp.float32),
    )(w_t)


def _rsqrt16(ss):
    bits = lax.bitcast_convert_type(ss, jnp.int32)
    y = lax.bitcast_convert_type(
        jnp.int32(0x5F3759DF) - lax.shift_right_arithmetic(bits, 1),
        jnp.float32)
    for _ in range(3):
        y = y * (1.5 - 0.5 * ss * y * y)
    return y


def _sc_kernel(w_hbm, idx_hbm, out_hbm, idx_v, rows_a, rows_b, outt,
               sem_a, sem_b, sem_oa, sem_ob):
    wid = lax.axis_index("s") * NC + lax.axis_index("c")
    base = wid * B_PER_W
    iota = lax.iota(jnp.int32, L)

    pltpu.sync_copy(idx_hbm.at[pl.ds(base, B_PER_W)], idx_v)

    def remap_idx(i, carry):
        v = idx_v[pl.ds(i * L, L)]
        idx_v[pl.ds(i * L, L)] = (
            lax.shift_left(lax.shift_right_logical(v, 10), 10)
            | lax.shift_left(v & 127, 3)
            | (lax.shift_right_logical(v, 7) & 7))
        return carry

    lax.fori_loop(0, B_PER_W // L, remap_idx, 0, unroll=False)

    rows = (rows_a, rows_b)
    gsems = (sem_a, sem_b)
    osems = (sem_oa, sem_ob)

    def gather(sc):
        b = sc % 2
        return pltpu.async_copy(
            w_hbm.at[idx_v.at[pl.ds(sc * C, C)]], rows[b], gsems[b])

    def compute_sc(sc):
        cur = sc % 2
        buf = rows[cur]

        def body(blk, carry):
            for g in range(GROUPS):
                row_ids = blk * BLK + g * L + iota
                cols = [
                    plsc.load_gather(
                        buf, [row_ids, jnp.full((L,), j, jnp.int32)])
                    for j in range(M)
                ]
                ss = cols[0] * cols[0]
                for j in range(1, M):
                    ss = ss + cols[j] * cols[j]
                scale = jnp.where(ss > MAX_NORM_SQ, MAX_NORM * _rsqrt16(ss),
                                  jnp.float32(1.0))
                for j in range(M):
                    outt[cur, j // 8, blk,
                         pl.ds((j % 8) * BLK + g * L, L)] = cols[j] * scale
            return carry

        lax.fori_loop(0, BLK_SC, body, 0, unroll=False)

    def emit_out(sc):
        cur = sc % 2
        handles = []
        for blk in range(BLK_SC):
            gblk = wid * NBLK_W + sc * BLK_SC + blk
            f26 = gblk // NBLK_B
            cpos = gblk % NBLK_B
            for band in range(2):
                off = (f26 * 256 + band * NBLK_B + cpos) * 1024
                off = pl.multiple_of(off, 1024)
                handles.append(pltpu.async_copy(
                    outt.at[cur, band, blk],
                    out_hbm.at[pl.ds(off, 1024)], osems[cur]))
        return handles

    pending_out = [None, None]
    ghandles = {0: gather(0)}
    for sc in range(NSC):
        b = sc % 2
        if sc + 1 < NSC:
            ghandles[sc + 1] = gather(sc + 1)
        ghandles.pop(sc).wait()
        if pending_out[b] is not None:
            for h in pending_out[b]:
                h.wait()
        compute_sc(sc)
        pending_out[b] = emit_out(sc)
    for par in pending_out:
        if par is not None:
            for h in par:
                h.wait()


@jax.jit
def _run(idx_flat, w16):
    mesh = plsc.VectorSubcoreMesh(core_axis_name="c", subcore_axis_name="s")
    f = pl.kernel(
        _sc_kernel,
        out_type=jax.ShapeDtypeStruct((OUT_LEN,), jnp.float32),
        mesh=mesh,
        compiler_params=pltpu.CompilerParams(
            needs_layout_passes=False, use_tc_tiling_on_sc=False),
        scratch_types=[
            pltpu.VMEM((B_PER_W,), jnp.int32),
            pltpu.VMEM((C, M), jnp.float32),
            pltpu.VMEM((C, M), jnp.float32),
            pltpu.VMEM((2, 2, BLK_SC, 1024), jnp.float32),
            pltpu.SemaphoreType.DMA,
            pltpu.SemaphoreType.DMA,
            pltpu.SemaphoreType.DMA,
            pltpu.SemaphoreType.DMA,
        ],
    )
    return f(w16, idx_flat)


def kernel(x, weight):
    idx_flat = x.T.reshape(-1).astype(jnp.int32)
    w_blk = _to_blocked(weight.T)
    w16 = w_blk.reshape(W16_ROWS * M).reshape(W16_ROWS, M)
    out_flat = _run(idx_flat, w16)
    out5 = out_flat.reshape(F, 2, NBLK_B, 8, BLK)
    return out5.transpose(2, 4, 0, 1, 3).reshape(B, F, M)
